# baseline (speedup 1.0000x reference)
import functools

import jax
import jax.numpy as jnp
from jax.experimental import pallas as pl

_INPUT_SIZE = 128
_T = 16
_V = 64


def _decode_block(x_ref, w_ref, seq_ref, score_ref, *, bn):
    logits = jnp.dot(x_ref[...], w_ref[...], preferred_element_type=jnp.float32)
    l3 = logits.reshape(bn, _T, _V)
    maxv = jnp.max(l3, axis=-1)
    idx = jnp.argmax(l3, axis=-1).astype(jnp.int32)
    seq_ref[...] = idx
    score_ref[...] = jnp.sum(maxv, axis=-1, keepdims=True)


@functools.partial(jax.jit, static_argnames=())
def kernel(observation, W):
    batch, d = observation.shape
    ipo = d // _INPUT_SIZE
    n = batch * ipo
    x = observation.reshape(n, _INPUT_SIZE)
    w = W.reshape(_INPUT_SIZE, _T * _V)

    bn = 2048
    grid = (n // bn,)
    seqs, scores = pl.pallas_call(
        functools.partial(_decode_block, bn=bn),
        grid=grid,
        in_specs=[
            pl.BlockSpec((bn, _INPUT_SIZE), lambda i: (i, 0)),
            pl.BlockSpec((_INPUT_SIZE, _T * _V), lambda i: (0, 0)),
        ],
        out_specs=[
            pl.BlockSpec((bn, _T), lambda i: (i, 0)),
            pl.BlockSpec((bn, 1), lambda i: (i, 0)),
        ],
        out_shape=[
            jax.ShapeDtypeStruct((n, _T), jnp.int32),
            jax.ShapeDtypeStruct((n, 1), jnp.float32),
        ],
    )(x, w)

    seq_supp_batch = seqs.reshape(batch, ipo, _T)
    score_batch = scores.reshape(batch, ipo)
    length_supp_batch = jnp.full((batch, ipo), _T, dtype=jnp.int32)
    return seq_supp_batch, length_supp_batch, score_batch


# speedup vs baseline: 3.4395x; 3.4395x over previous
import functools

import jax
import jax.numpy as jnp
from jax.experimental import pallas as pl

_INPUT_SIZE = 128
_T = 16
_V = 64


def _decode_block(x_ref, a_ref, seq_ref, score_ref, *, bn):
    lt = jax.lax.dot_general(
        a_ref[...], x_ref[...],
        dimension_numbers=(((1,), (1,)), ((), ())),
        preferred_element_type=jnp.float32,
    )
    l3 = lt.reshape(_T, _V, bn)
    maxv = jnp.max(l3, axis=1)
    hit = l3 == maxv[:, None, :]
    vio = jax.lax.broadcasted_iota(jnp.int32, (_T, _V, bn), 1)
    idx = jnp.min(jnp.where(hit, vio, _V), axis=1).astype(jnp.int32)
    seq_ref[...] = idx
    score_ref[...] = jnp.sum(maxv, axis=0, keepdims=True)


def kernel(observation, W):
    batch, d = observation.shape
    ipo = d // _INPUT_SIZE
    n = batch * ipo
    x = observation.reshape(n, _INPUT_SIZE)
    a = W.reshape(_INPUT_SIZE, _T * _V).T

    bn = 2048
    grid = (n // bn,)
    seqs_t, scores_t = pl.pallas_call(
        functools.partial(_decode_block, bn=bn),
        grid=grid,
        in_specs=[
            pl.BlockSpec((bn, _INPUT_SIZE), lambda i: (i, 0)),
            pl.BlockSpec((_T * _V, _INPUT_SIZE), lambda i: (0, 0)),
        ],
        out_specs=[
            pl.BlockSpec((_T, bn), lambda i: (0, i)),
            pl.BlockSpec((1, bn), lambda i: (0, i)),
        ],
        out_shape=[
            jax.ShapeDtypeStruct((_T, n), jnp.int32),
            jax.ShapeDtypeStruct((1, n), jnp.float32),
        ],
    )(x, a)

    seq_supp_batch = seqs_t.T.reshape(batch, ipo, _T)
    score_batch = scores_t.reshape(batch, ipo)
    length_supp_batch = jnp.full((batch, ipo), _T, dtype=jnp.int32)
    return seq_supp_batch, length_supp_batch, score_batch
